# baseline (speedup 1.0000x reference)
import numpy as np

import jax
import jax.numpy as jnp
from jax import lax
from jax.experimental import pallas as pl
from jax.experimental.pallas import tpu as pltpu

ROWS = 128
COLS = 8192
BLK = 256
CH = 32
NBLK = COLS // BLK
NCH = ROWS // CH

_TRI = np.triu(np.ones((BLK, BLK), np.float32)).astype(jnp.bfloat16)


def _compute(xs, tri, o_buf):
    xb = xs.astype(jnp.bfloat16)
    stacked = jnp.concatenate(
        [xb[:, b * BLK:(b + 1) * BLK] for b in range(NBLK)], axis=0)
    ys = lax.dot_general(stacked, tri, (((1,), (0,)), ((), ())),
                         preferred_element_type=jnp.float32)
    off = jnp.zeros((CH, 1), jnp.float32)
    for b in range(NBLK):
        yb = ys[b * CH:(b + 1) * CH, :]
        o_buf[:, b * BLK:(b + 1) * BLK] = yb + off
        off = off + yb[:, BLK - 1:BLK]
    return off


def _body(x_hbm, tri_ref, o_hbm, *refs):
    ibufs, obufs = refs[:NCH], refs[NCH:2 * NCH]
    isems = refs[2 * NCH:3 * NCH]
    osems = refs[3 * NCH:4 * NCH]

    def in_copy(c):
        return pltpu.make_async_copy(
            x_hbm.at[pl.ds(c * CH, CH)], ibufs[c], isems[c])

    def out_copy(c):
        return pltpu.make_async_copy(
            obufs[c], o_hbm.at[pl.ds(c * CH, CH)], osems[c])

    tri = tri_ref[...]
    in_copy(0).start()
    in_copy(1).start()
    for c in range(NCH):
        in_copy(c).wait()
        if c + 2 < NCH:
            in_copy(c + 2).start()
        _compute(ibufs[c][...], tri, obufs[c])
        out_copy(c).start()
    for c in range(NCH):
        out_copy(c).wait()


def kernel(x):
    return pl.pallas_call(
        _body,
        in_specs=[pl.BlockSpec(memory_space=pltpu.HBM),
                  pl.BlockSpec(memory_space=pltpu.VMEM)],
        out_specs=pl.BlockSpec(memory_space=pltpu.HBM),
        out_shape=jax.ShapeDtypeStruct((ROWS, COLS), jnp.float32),
        scratch_shapes=(
            [pltpu.VMEM((CH, COLS), jnp.float32)] * (2 * NCH)
            + [pltpu.SemaphoreType.DMA] * (2 * NCH)
        ),
    )(x, jnp.asarray(_TRI))
